# Initial kernel scaffold; baseline (speedup 1.0000x reference)
import jax
import jax.numpy as jnp
from jax.experimental import pallas as pl


def kernel(x):
    raise NotImplementedError("write your pallas kernel here")


# speedup vs baseline: 2.0965x; 2.0965x over previous
import jax
import jax.numpy as jnp
from jax.experimental import pallas as pl
from jax.experimental.pallas import tpu as pltpu

_N, _H, _W, _C = 32, 56, 56, 256
_OH = _OW = 29
_CB = 128
_NC = _C // _CB


def _diff(img):
    a = img[2:60] - img[0:58]
    b = img[2:60, 2:60] - img[2:60, 0:58]
    return jnp.abs(a[:, 2:60]) + jnp.abs(b)


def _dsum_kernel(x_ref, out_ref):
    img = jnp.pad(x_ref[0], ((2, 2), (2, 2), (0, 0)))
    d = _diff(img)
    dv = d.reshape(29, 2, 58, _CB)
    dev = dv[:, 0]
    dod = dv[:, 1]
    hsum = (2.0 * jnp.sum(dev[0:28], axis=0)
            + jnp.sum(dod[0:28], axis=0)
            + dev[28])
    j = jax.lax.broadcasted_iota(jnp.int32, (58, 1), 0)
    w = jnp.where(j == 57, 0.0,
                  jnp.where(j == 56, 1.0,
                            jnp.where(j % 2 == 0, 2.0, 1.0)))
    out_ref[0, 0] = jnp.sum(hsum * w, axis=0, keepdims=True)


def _pool_kernel(thresh_ref, x_ref, out_ref, rm_s, rs_s, cen_s):
    t = thresh_ref[0, 0]
    img = jnp.pad(x_ref[0], ((2, 2), (2, 2), (0, 0)))

    iv = img.reshape(30, 2, 60, _CB)
    ev = iv[:, 0]
    od = iv[:, 1]
    lA, lB, lC = ev[0:29], od[0:29], ev[1:30]
    rm_s[...] = jnp.maximum(jnp.maximum(lA, lB), lC)
    rs_s[...] = lA + lB + lC

    mA = rm_s[:, pl.ds(0, 29, 2), :]
    mB = rm_s[:, pl.ds(1, 29, 2), :]
    mC = rm_s[:, pl.ds(2, 29, 2), :]
    pmax = jnp.maximum(jnp.maximum(mA, mB), mC)
    sA = rs_s[:, pl.ds(0, 29, 2), :]
    sB = rs_s[:, pl.ds(1, 29, 2), :]
    sC = rs_s[:, pl.ds(2, 29, 2), :]
    pmean = (sA + sB + sC) * (1.0 / 9.0)

    d = _diff(img)
    cr = d.reshape(29, 2, 58, _CB)[:, 1]
    cen_s[...] = cr[0:28]
    cc = cen_s[:, pl.ds(1, 28, 2), :]
    center = jnp.pad(cc, ((1, 0), (1, 0), (0, 0)))

    out_ref[0] = jnp.where(center > t, pmax, pmean)


def kernel(x):
    partial = pl.pallas_call(
        _dsum_kernel,
        grid=(_N, _NC),
        in_specs=[pl.BlockSpec((1, _H, _W, _CB), lambda i, j: (i, 0, 0, j))],
        out_specs=pl.BlockSpec((1, 1, 1, _CB), lambda i, j: (i, j, 0, 0)),
        out_shape=jax.ShapeDtypeStruct((_N, _NC, 1, _CB), jnp.float32),
        compiler_params=pltpu.CompilerParams(
            dimension_semantics=("parallel", "parallel")),
    )(x)
    thresh = (jnp.sum(partial) / (9.0 * _N * _C * _OH * _OW)).reshape(1, 1)
    return pl.pallas_call(
        _pool_kernel,
        grid=(_N, _NC),
        in_specs=[
            pl.BlockSpec(memory_space=pltpu.SMEM),
            pl.BlockSpec((1, _H, _W, _CB), lambda i, j: (i, 0, 0, j)),
        ],
        out_specs=pl.BlockSpec((1, _OH, _OW, _CB), lambda i, j: (i, 0, 0, j)),
        out_shape=jax.ShapeDtypeStruct((_N, _OH, _OW, _C), jnp.float32),
        scratch_shapes=[
            pltpu.VMEM((_OH, 60, _CB), jnp.float32),
            pltpu.VMEM((_OH, 60, _CB), jnp.float32),
            pltpu.VMEM((28, 58, _CB), jnp.float32),
        ],
        compiler_params=pltpu.CompilerParams(
            dimension_semantics=("parallel", "parallel")),
    )(thresh, x)


# speedup vs baseline: 2.4397x; 1.1637x over previous
import jax
import jax.numpy as jnp
from jax.experimental import pallas as pl
from jax.experimental.pallas import tpu as pltpu

_N, _H, _W, _C = 32, 56, 56, 256
_OH = _OW = 29
_CB = 128


def _diff(img):
    a = img[2:60] - img[0:58]
    b = img[2:60, 2:60] - img[2:60, 0:58]
    return jnp.abs(a[:, 2:60]) + jnp.abs(b)


def _dsum_kernel(x_ref, out_ref):
    img = jnp.pad(x_ref[0], ((2, 2), (2, 2), (0, 0)))
    d = _diff(img)
    dv = d.reshape(29, 2, 58, _C)
    dev = dv[:, 0]
    dod = dv[:, 1]
    hsum = (2.0 * jnp.sum(dev[0:28], axis=0)
            + jnp.sum(dod[0:28], axis=0)
            + dev[28])
    j = jax.lax.broadcasted_iota(jnp.int32, (58, 1), 0)
    w = jnp.where(j == 57, 0.0,
                  jnp.where(j == 56, 1.0,
                            jnp.where(j % 2 == 0, 2.0, 1.0)))
    out_ref[0, 0] = jnp.sum(hsum * w, axis=0, keepdims=True)


def _pool_kernel(thresh_ref, x_ref, out_ref, rm_s, rs_s, cen_s):
    t = thresh_ref[0, 0]
    for h in range(_C // _CB):
        sl = slice(h * _CB, (h + 1) * _CB)
        img = jnp.pad(x_ref[0, :, :, sl], ((2, 2), (2, 2), (0, 0)))

        iv = img.reshape(30, 2, 60, _CB)
        ev = iv[:, 0]
        od = iv[:, 1]
        lA, lB, lC = ev[0:29], od[0:29], ev[1:30]
        rm_s[...] = jnp.maximum(jnp.maximum(lA, lB), lC)
        rs_s[...] = lA + lB + lC

        mA = rm_s[:, pl.ds(0, 29, 2), :]
        mB = rm_s[:, pl.ds(1, 29, 2), :]
        mC = rm_s[:, pl.ds(2, 29, 2), :]
        pmax = jnp.maximum(jnp.maximum(mA, mB), mC)
        sA = rs_s[:, pl.ds(0, 29, 2), :]
        sB = rs_s[:, pl.ds(1, 29, 2), :]
        sC = rs_s[:, pl.ds(2, 29, 2), :]
        pmean = (sA + sB + sC) * (1.0 / 9.0)

        d = _diff(img)
        cr = d.reshape(29, 2, 58, _CB)[:, 1]
        cen_s[...] = cr[0:28]
        cc = cen_s[:, pl.ds(1, 28, 2), :]
        center = jnp.pad(cc, ((1, 0), (1, 0), (0, 0)))

        out_ref[0, :, :, sl] = jnp.where(center > t, pmax, pmean)


def kernel(x):
    partial = pl.pallas_call(
        _dsum_kernel,
        grid=(_N,),
        in_specs=[pl.BlockSpec((1, _H, _W, _C), lambda i: (i, 0, 0, 0))],
        out_specs=pl.BlockSpec((1, 1, 1, _C), lambda i: (i, 0, 0, 0)),
        out_shape=jax.ShapeDtypeStruct((_N, 1, 1, _C), jnp.float32),
        compiler_params=pltpu.CompilerParams(
            dimension_semantics=("parallel",)),
    )(x)
    thresh = (jnp.sum(partial) / (9.0 * _N * _C * _OH * _OW)).reshape(1, 1)
    return pl.pallas_call(
        _pool_kernel,
        grid=(_N,),
        in_specs=[
            pl.BlockSpec(memory_space=pltpu.SMEM),
            pl.BlockSpec((1, _H, _W, _C), lambda i: (i, 0, 0, 0)),
        ],
        out_specs=pl.BlockSpec((1, _OH, _OW, _C), lambda i: (i, 0, 0, 0)),
        out_shape=jax.ShapeDtypeStruct((_N, _OH, _OW, _C), jnp.float32),
        scratch_shapes=[
            pltpu.VMEM((_OH, 60, _CB), jnp.float32),
            pltpu.VMEM((_OH, 60, _CB), jnp.float32),
            pltpu.VMEM((28, 58, _CB), jnp.float32),
        ],
        compiler_params=pltpu.CompilerParams(
            dimension_semantics=("parallel",)),
    )(thresh, x)


# speedup vs baseline: 2.5152x; 1.0309x over previous
import jax
import jax.numpy as jnp
from jax.experimental import pallas as pl
from jax.experimental.pallas import tpu as pltpu

_N, _H, _W, _C = 32, 56, 56, 256
_OH = _OW = 29
_CB = 128
_INV9 = 1.0 / 9.0
_NDCOL = 9.0 * _N * _C * _OH * _OW
_PW = 66


def _dsum_kernel(x_ref, out_ref, img_s):
    img_s[0:2] = jnp.zeros((2, _PW, _C), jnp.float32)
    img_s[58:60] = jnp.zeros((2, _PW, _C), jnp.float32)
    img_s[2:58, 0:8] = jnp.zeros((56, 8, _C), jnp.float32)
    img_s[2:58, 64:66] = jnp.zeros((56, 2, _C), jnp.float32)
    img_s[2:58, 8:64] = x_ref[0]
    c2 = img_s[:, 8:66]
    c2r = c2[2:60]
    c0r = jnp.concatenate(
        [jnp.zeros((58, 2, _C), jnp.float32), c2r[:, 0:56]], axis=1)
    d = jnp.abs(c2r - c2[0:58]) + jnp.abs(c2r - c0r)
    dv = d.reshape(29, 2, 58, _C)
    dev = dv[:, 0]
    dod = dv[:, 1]
    hsum = (2.0 * jnp.sum(dev[0:28], axis=0)
            + jnp.sum(dod[0:28], axis=0)
            + dev[28])
    j = jax.lax.broadcasted_iota(jnp.int32, (58, 1), 0)
    w = jnp.where(j == 57, 0.0,
                  jnp.where(j == 56, 1.0,
                            jnp.where(j % 2 == 0, 2.0, 1.0)))
    out_ref[0, 0] = jnp.sum(hsum * w, axis=0, keepdims=True)


def _pool_kernel(part_ref, x_ref, out_ref, rm_s, rs_s, oi_s):
    t = jnp.sum(part_ref[...].reshape(_N, _C)) / _NDCOL
    for h in range(_C // _CB):
        sl = slice(h * _CB, (h + 1) * _CB)
        img = jnp.pad(x_ref[0, :, :, sl], ((2, 2), (2, 2), (0, 0)))

        iv = img.reshape(30, 2, 60, _CB)
        ev, od = iv[:, 0], iv[:, 1]
        lA, lB, lC = ev[0:29], od[0:29], ev[1:30]
        rm_s[...] = jnp.maximum(jnp.maximum(lA, lB), lC)
        rs_s[...] = lA + lB + lC

        mA = rm_s[:, pl.ds(0, 29, 2), :]
        mB = rm_s[:, pl.ds(1, 29, 2), :]
        mC = rm_s[:, pl.ds(2, 29, 2), :]
        pmax = jnp.maximum(jnp.maximum(mA, mB), mC)
        sA = rs_s[:, pl.ds(0, 29, 2), :]
        sB = rs_s[:, pl.ds(1, 29, 2), :]
        sC = rs_s[:, pl.ds(2, 29, 2), :]
        pmean = (sA + sB + sC) * _INV9

        oi_s[...] = od
        q3 = oi_s[:, pl.ds(3, 28, 2), :]
        q1 = oi_s[:, pl.ds(1, 28, 2), :]
        cen = (jnp.abs(q3[1:29] - q3[0:28])
               + jnp.abs(q3[1:29] - q1[1:29]))
        center = jnp.pad(cen, ((1, 0), (1, 0), (0, 0)))

        out_ref[0, :, :, sl] = jnp.where(center > t, pmax, pmean)


def kernel(x):
    partial = pl.pallas_call(
        _dsum_kernel,
        grid=(_N,),
        in_specs=[pl.BlockSpec((1, _H, _W, _C), lambda i: (i, 0, 0, 0))],
        out_specs=pl.BlockSpec((1, 1, 1, _C), lambda i: (i, 0, 0, 0)),
        out_shape=jax.ShapeDtypeStruct((_N, 1, 1, _C), jnp.float32),
        scratch_shapes=[pltpu.VMEM((60, _PW, _C), jnp.float32)],
        compiler_params=pltpu.CompilerParams(
            dimension_semantics=("parallel",)),
    )(x)
    return pl.pallas_call(
        _pool_kernel,
        grid=(_N,),
        in_specs=[
            pl.BlockSpec((_N, 1, 1, _C), lambda i: (0, 0, 0, 0)),
            pl.BlockSpec((1, _H, _W, _C), lambda i: (i, 0, 0, 0)),
        ],
        out_specs=pl.BlockSpec((1, _OH, _OW, _C), lambda i: (i, 0, 0, 0)),
        out_shape=jax.ShapeDtypeStruct((_N, _OH, _OW, _C), jnp.float32),
        scratch_shapes=[
            pltpu.VMEM((_OH, 60, _CB), jnp.float32),
            pltpu.VMEM((_OH, 60, _CB), jnp.float32),
            pltpu.VMEM((30, 60, _CB), jnp.float32),
        ],
        compiler_params=pltpu.CompilerParams(
            dimension_semantics=("parallel",)),
    )(partial, x)
